# baseline (speedup 1.0000x reference)
import functools

import jax
import jax.numpy as jnp
from jax import lax
from jax.experimental import pallas as pl
from jax.experimental.pallas import tpu as pltpu
from jax.experimental.pallas import tpu_sc as plsc

B = 16384
D3 = 256
_GATHER_CHUNK = 128
_NSEG = 4
_BS = B // _NSEG


def _leaky(x):
    return jnp.where(x > 0, x, 0.01 * x)


def _dot_tn(a, b):
    return lax.dot_general(a, b, (((0,), (0,)), ((), ())),
                           preferred_element_type=jnp.float32)


def _dot_nn(a, b):
    return jnp.dot(a, b, preferred_element_type=jnp.float32)


def _prep_body(c3_ref, idx_ref, canvas_ref):
    idx_ref[...] = c3_ref[...].reshape(128, 128)
    canvas_ref[...] = jnp.zeros(canvas_ref.shape, canvas_ref.dtype)


def _tc_prep(cat3):
    return pl.pallas_call(
        _prep_body,
        grid=(1,),
        in_specs=[pl.BlockSpec((1, B), lambda i: (0, 0))],
        out_specs=[
            pl.BlockSpec((128, 128), lambda i: (0, 0)),
            pl.BlockSpec((8, 128), lambda i: (0, 0)),
        ],
        out_shape=[
            jax.ShapeDtypeStruct((128, 128), jnp.int32),
            jax.ShapeDtypeStruct((B, 128), jnp.float32),
        ],
    )(cat3)


def _sc_gather_seg(table, idx2d, seg):
    info = plsc.get_sparse_core_info()
    nw = info.num_cores * info.num_subcores
    b_per_w = _BS // nw
    n_chunks = b_per_w // _GATHER_CHUNK

    mesh = plsc.VectorSubcoreMesh(core_axis_name="c", subcore_axis_name="s")

    @functools.partial(
        pl.kernel,
        mesh=mesh,
        out_type=jax.ShapeDtypeStruct((_BS, D3), jnp.float32),
        scratch_types=[
            pltpu.VMEM((8, _GATHER_CHUNK), jnp.int32),
            pltpu.VMEM((_GATHER_CHUNK, D3), jnp.float32),
            pltpu.VMEM((_GATHER_CHUNK, D3), jnp.float32),
            pltpu.SemaphoreType.DMA,
            pltpu.SemaphoreType.DMA,
            pltpu.SemaphoreType.DMA,
            pltpu.SemaphoreType.DMA,
        ],
    )
    def gather_k(table_hbm, idx_hbm, out_hbm, idx_v, rows_a, rows_b,
                 gsem_a, gsem_b, osem_a, osem_b):
        wid = lax.axis_index("s") * info.num_cores + lax.axis_index("c")
        base = wid * b_per_w
        idx_row0 = (seg * _BS + base) // _GATHER_CHUNK
        slab0 = pl.multiple_of((idx_row0 // 8) * 8, 8)
        inner = idx_row0 - (idx_row0 // 8) * 8
        rows = (rows_a, rows_b)
        gsem = (gsem_a, gsem_b)
        osem = (osem_a, osem_b)
        pltpu.sync_copy(idx_hbm.at[pl.ds(slab0, 8)], idx_v)
        out_copies = [None] * n_chunks
        prev = None
        for c in range(n_chunks):
            if c >= 2:
                out_copies[c - 2].wait()
            g = pltpu.async_copy(table_hbm.at[idx_v.at[inner + c]],
                                 rows[c % 2], gsem[c % 2])
            if prev is not None:
                pc, pg = prev
                pg.wait()
                out_copies[pc] = pltpu.async_copy(
                    rows[pc % 2],
                    out_hbm.at[pl.ds(base + pc * _GATHER_CHUNK, _GATHER_CHUNK)],
                    osem[pc % 2])
            prev = (c, g)
        pc, pg = prev
        pg.wait()
        out_copies[pc] = pltpu.async_copy(
            rows[pc % 2],
            out_hbm.at[pl.ds(base + pc * _GATHER_CHUNK, _GATHER_CHUNK)],
            osem[pc % 2])
        if n_chunks >= 2:
            out_copies[n_chunks - 2].wait()
        out_copies[n_chunks - 1].wait()

    return gather_k(table, idx2d)


def _final_body(canvas_ref, numT_ref, c1_ref, c2_ref, e3_ref,
                W1T_ref, b1_ref, W2T_ref, b2_ref, E1_ref, E2_ref, W3T_ref,
                b3_ref, out_ref):
    blk = e3_ref.shape[0]
    v = _leaky(_dot_tn(numT_ref[...], W1T_ref[...]) + b1_ref[...])
    v = _leaky(_dot_nn(v, W2T_ref[...]) + b2_ref[...])
    acc = _dot_nn(v, W3T_ref[0:128, :])
    acc += _dot_nn(e3_ref[...], W3T_ref[169:425, :])
    oh1t = (c1_ref[...] == lax.broadcasted_iota(jnp.int32, (4, blk), 0)
            ).astype(jnp.float32)
    e1 = _dot_tn(oh1t, E1_ref[...])
    acc += _dot_nn(e1, W3T_ref[128:144, :])
    oh2t = (c2_ref[...] == lax.broadcasted_iota(jnp.int32, (5, blk), 0)
            ).astype(jnp.float32)
    e2 = _dot_tn(oh2t, E2_ref[...])
    acc += _dot_nn(e2, W3T_ref[144:169, :])
    acc += b3_ref[...]
    out_ref[...] = _leaky(acc)


def _tc_final_seg(seg, canvas, numericT, c1, c2, e3_seg,
                  W1T, b1, W2T, b2, E1, E2, W3T, b3, blk=2048):
    grid = _BS // blk
    off = seg * grid

    def full(shape):
        return pl.BlockSpec(shape, lambda i: tuple(0 for _ in shape))

    return pl.pallas_call(
        _final_body,
        grid=(grid,),
        in_specs=[
            pl.BlockSpec((blk, 128), lambda i: (i + off, 0)),
            pl.BlockSpec((3, blk), lambda i: (0, i + off)),
            pl.BlockSpec((1, blk), lambda i: (0, i + off)),
            pl.BlockSpec((1, blk), lambda i: (0, i + off)),
            pl.BlockSpec((blk, D3), lambda i: (i, 0)),
            full((3, 64)),
            full((64,)),
            full((64, 128)),
            full((128,)),
            full((4, 16)),
            full((5, 25)),
            full((425, 128)),
            full((128,)),
        ],
        out_specs=pl.BlockSpec((blk, 128), lambda i: (i + off, 0)),
        out_shape=jax.ShapeDtypeStruct((B, 128), jnp.float32),
        input_output_aliases={0: 0},
    )(canvas, numericT, c1, c2, e3_seg, W1T, b1, W2T, b2, E1, E2, W3T, b3)


def kernel(numeric, cat1, cat2, cat3, W1, b1, W2, b2, E1, E2, E3, W3, b3):
    idx2d, canvas = _tc_prep(cat3.astype(jnp.int32))
    e3_segs = [_sc_gather_seg(E3, idx2d, s) for s in range(_NSEG)]

    c1 = cat1.astype(jnp.int32)
    c2 = cat2.astype(jnp.int32)
    y = canvas
    for s in range(_NSEG):
        y = _tc_final_seg(s, y, numeric.T, c1, c2, e3_segs[s],
                          W1.T, b1, W2.T, b2, E1, E2, W3.T, b3)
    return y


# speedup vs baseline: 1.1936x; 1.1936x over previous
import functools

import jax
import jax.numpy as jnp
from jax import lax
from jax.experimental import pallas as pl
from jax.experimental.pallas import tpu as pltpu
from jax.experimental.pallas import tpu_sc as plsc

B = 16384
D3 = 256
_GATHER_CHUNK = 128
_NSEG = 2
_BS = B // _NSEG


def _leaky(x):
    return jnp.where(x > 0, x, 0.01 * x)


def _dot_nt(a, b):
    return lax.dot_general(a, b, (((1,), (1,)), ((), ())),
                           preferred_element_type=jnp.float32)


def _dot_tn(a, b):
    return lax.dot_general(a, b, (((0,), (0,)), ((), ())),
                           preferred_element_type=jnp.float32)


def _dot_nn(a, b):
    return jnp.dot(a, b, preferred_element_type=jnp.float32)


def _prep_body(c3_ref, idx_ref, canvas_ref):
    idx_ref[...] = c3_ref[...].reshape(128, 128)
    canvas_ref[...] = jnp.zeros(canvas_ref.shape, canvas_ref.dtype)


def _tc_prep(cat3):
    return pl.pallas_call(
        _prep_body,
        grid=(1,),
        in_specs=[pl.BlockSpec((1, B), lambda i: (0, 0))],
        out_specs=[
            pl.BlockSpec((128, 128), lambda i: (0, 0)),
            pl.BlockSpec((8, 128), lambda i: (0, 0)),
        ],
        out_shape=[
            jax.ShapeDtypeStruct((128, 128), jnp.int32),
            jax.ShapeDtypeStruct((B, 128), jnp.float32),
        ],
    )(cat3)


def _sc_gather_seg(table, idx2d, seg):
    info = plsc.get_sparse_core_info()
    nw = info.num_cores * info.num_subcores
    b_per_w = _BS // nw
    n_chunks = b_per_w // _GATHER_CHUNK
    rows_per_w = b_per_w // _GATHER_CHUNK

    mesh = plsc.VectorSubcoreMesh(core_axis_name="c", subcore_axis_name="s")

    @functools.partial(
        pl.kernel,
        mesh=mesh,
        out_type=jax.ShapeDtypeStruct((_BS, D3), jnp.float32),
        scratch_types=[
            pltpu.VMEM((8, _GATHER_CHUNK), jnp.int32),
            pltpu.VMEM((_GATHER_CHUNK, D3), jnp.float32),
            pltpu.VMEM((_GATHER_CHUNK, D3), jnp.float32),
            pltpu.SemaphoreType.DMA,
            pltpu.SemaphoreType.DMA,
            pltpu.SemaphoreType.DMA,
            pltpu.SemaphoreType.DMA,
        ],
    )
    def gather_k(table_hbm, idx_hbm, out_hbm, idx_v, rows_a, rows_b,
                 gsem_a, gsem_b, osem_a, osem_b):
        wid = lax.axis_index("s") * info.num_cores + lax.axis_index("c")
        base = wid * b_per_w
        idx_row0 = (seg * _BS + base) // _GATHER_CHUNK
        slab0 = pl.multiple_of((idx_row0 // 8) * 8, 8)
        inner = idx_row0 - (idx_row0 // 8) * 8
        rows = (rows_a, rows_b)
        gsem = (gsem_a, gsem_b)
        osem = (osem_a, osem_b)
        pltpu.sync_copy(idx_hbm.at[pl.ds(slab0, 8)], idx_v)
        out_copies = [None] * n_chunks
        prev = None
        for c in range(n_chunks):
            if c >= 2:
                out_copies[c - 2].wait()
            g = pltpu.async_copy(table_hbm.at[idx_v.at[inner + c]],
                                 rows[c % 2], gsem[c % 2])
            if prev is not None:
                pc, pg = prev
                pg.wait()
                out_copies[pc] = pltpu.async_copy(
                    rows[pc % 2],
                    out_hbm.at[pl.ds(base + pc * _GATHER_CHUNK, _GATHER_CHUNK)],
                    osem[pc % 2])
            prev = (c, g)
        pc, pg = prev
        pg.wait()
        out_copies[pc] = pltpu.async_copy(
            rows[pc % 2],
            out_hbm.at[pl.ds(base + pc * _GATHER_CHUNK, _GATHER_CHUNK)],
            osem[pc % 2])
        if n_chunks >= 2:
            out_copies[n_chunks - 2].wait()
        out_copies[n_chunks - 1].wait()

    return gather_k(table, idx2d)


def _final_body(canvas_ref, numT_ref, c1_ref, c2_ref, e3_ref,
                W1T_ref, b1_ref, W2T_ref, b2_ref, E1_ref, E2_ref, W3T_ref,
                b3_ref, out_ref):
    del canvas_ref
    blk = e3_ref.shape[0]
    v = _leaky(_dot_tn(numT_ref[...], W1T_ref[...]) + b1_ref[...])
    v = _leaky(_dot_nn(v, W2T_ref[...]) + b2_ref[...])
    acc = _dot_nn(v, W3T_ref[0:128, :])
    acc += _dot_nn(e3_ref[...], W3T_ref[169:425, :])
    oh1t = (c1_ref[...] == lax.broadcasted_iota(jnp.int32, (4, blk), 0)
            ).astype(jnp.float32)
    e1 = _dot_tn(oh1t, E1_ref[...])
    acc += _dot_nn(e1, W3T_ref[128:144, :])
    oh2t = (c2_ref[...] == lax.broadcasted_iota(jnp.int32, (5, blk), 0)
            ).astype(jnp.float32)
    e2 = _dot_tn(oh2t, E2_ref[...])
    acc += _dot_nn(e2, W3T_ref[144:169, :])
    acc += b3_ref[...]
    out_ref[...] = _leaky(acc)


def _tc_final_seg(seg, canvas, numericT, c1, c2, e3_seg,
                  W1T, b1, W2T, b2, E1, E2, W3T, b3, blk=4096):
    grid = _BS // blk
    off = seg * grid

    def full(shape):
        return pl.BlockSpec(shape, lambda i: tuple(0 for _ in shape))

    return pl.pallas_call(
        _final_body,
        grid=(grid,),
        in_specs=[
            pl.BlockSpec((8, 128), lambda i: (0, 0)),
            pl.BlockSpec((3, blk), lambda i: (0, i + off)),
            pl.BlockSpec((1, blk), lambda i: (0, i + off)),
            pl.BlockSpec((1, blk), lambda i: (0, i + off)),
            pl.BlockSpec((blk, D3), lambda i: (i, 0)),
            full((3, 64)),
            full((64,)),
            full((64, 128)),
            full((128,)),
            full((4, 16)),
            full((5, 25)),
            full((425, 128)),
            full((128,)),
        ],
        out_specs=pl.BlockSpec((blk, 128), lambda i: (i + off, 0)),
        out_shape=jax.ShapeDtypeStruct((B, 128), jnp.float32),
        input_output_aliases={0: 0},
    )(canvas, numericT, c1, c2, e3_seg, W1T, b1, W2T, b2, E1, E2, W3T, b3)


def kernel(numeric, cat1, cat2, cat3, W1, b1, W2, b2, E1, E2, E3, W3, b3):
    idx2d, canvas = _tc_prep(cat3.astype(jnp.int32))
    e3_segs = [_sc_gather_seg(E3, idx2d, s) for s in range(_NSEG)]

    c1 = cat1.astype(jnp.int32)
    c2 = cat2.astype(jnp.int32)
    y = canvas
    for s in range(_NSEG):
        y = _tc_final_seg(s, y, numeric.T, c1, c2, e3_segs[s],
                          W1.T, b1, W2.T, b2, E1, E2, W3.T, b3)
    return y
